# baseline (speedup 1.0000x reference)
import jax
import jax.numpy as jnp
from jax.experimental import pallas as pl
from jax.experimental.pallas import tpu as pltpu

_T_TILE = 128
_U = 8
_NS = 4
_KA = 256


def _lstm_tile_kernel(x_ref, waug_ref, out_ref, hn_ref, cn_ref, scr_ref,
                      aug_ref, ob_ref, sem):
    Bb, H = hn_ref.shape
    H3 = 3 * H
    sr = Bb // _NS
    bid = pl.program_id(0)
    tid = pl.program_id(1)
    n_tiles = pl.num_programs(1)
    nchunks = _T_TILE // _U

    def out_dma(slot, ci):
        return pltpu.make_async_copy(
            ob_ref.at[slot],
            out_ref.at[pl.ds(bid * Bb, Bb),
                       pl.ds(tid * _T_TILE + ci * _U, _U), :],
            sem.at[slot])

    def wait_slot(slot):
        pltpu.make_async_copy(ob_ref.at[slot], ob_ref.at[slot],
                              sem.at[slot]).wait()

    @pl.when(tid == 0)
    def _init():
        cn_ref[...] = jnp.zeros_like(cn_ref)
        aug_ref[...] = jnp.zeros_like(aug_ref)
        aug_ref[0, :, H + 1:H + 2] = jnp.ones((Bb, 1), jnp.bfloat16)
        aug_ref[1, :, H + 1:H + 2] = jnp.ones((Bb, 1), jnp.bfloat16)

    waug = waug_ref[...]
    xb = x_ref[...].astype(jnp.bfloat16)

    aug_ref[0, :, H:H + 1] = xb[:, 0:1]

    cs = [cn_ref[k * sr:(k + 1) * sr, :] for k in range(_NS)]
    hs32 = [None] * _NS

    for ci in range(nchunks):
        for j in range(_U):
            t = ci * _U + j
            slot, nslot = t % 2, (t + 1) % 2
            if t + 1 < _T_TILE:
                aug_ref[nslot, :, H:H + 1] = xb[:, t + 1:t + 2]
            for k in range(_NS):
                r0 = k * sr
                gates = jnp.dot(aug_ref[slot, r0:r0 + sr, :], waug,
                                preferred_element_type=jnp.float32)
                sig = jnp.tanh(gates[:, :H3]) * 0.5 + 0.5
                g_gate = jnp.tanh(gates[:, H3:])
                c = sig[:, H:2 * H] * cs[k] + sig[:, :H] * g_gate
                h32 = sig[:, 2 * H:H3] * jnp.tanh(c)
                cs[k] = c
                hs32[k] = h32
                aug_ref[nslot, r0:r0 + sr, 0:H] = h32.astype(jnp.bfloat16)
                scr_ref[r0:r0 + sr, j * H:(j + 1) * H] = h32
        dslot = ci % 2
        if ci >= 2:
            wait_slot(dslot)
        else:
            @pl.when(tid > 0)
            def _():
                wait_slot(dslot)
        rg = min(32, Bb)
        for r in range(0, Bb, rg):
            ob_ref[dslot, r:r + rg, :, :] = (
                scr_ref[r:r + rg, :].reshape(rg, _U, H))
        out_dma(dslot, ci).start()

    for k in range(_NS):
        hn_ref[k * sr:(k + 1) * sr, :] = hs32[k]
        cn_ref[k * sr:(k + 1) * sr, :] = cs[k]

    @pl.when(tid == n_tiles - 1)
    def _drain():
        wait_slot((nchunks - 2) % 2)
        wait_slot((nchunks - 1) % 2)


def kernel(x, w_ih, w_hh, b_ih, b_hh):
    B, T = x.shape
    H = w_hh.shape[1]

    def perm_gates(a, axis):
        i, f, g, o = jnp.split(a.astype(jnp.float32), 4, axis=axis)
        return jnp.concatenate([i, f, o, g], axis=axis)

    whh_t = perm_gates(jnp.transpose(w_hh), axis=1)
    wih_row = perm_gates(w_ih.reshape(1, 4 * H), axis=1)
    bias = perm_gates((b_ih + b_hh).reshape(1, 4 * H), axis=1)
    waug = jnp.concatenate(
        [whh_t, wih_row, bias,
         jnp.zeros((_KA - H - 2, 4 * H), jnp.float32)], axis=0)
    col_scale = jnp.concatenate([jnp.full((1, 3 * H), 0.5, jnp.float32),
                                 jnp.ones((1, H), jnp.float32)], axis=1)
    waug = (waug * col_scale).astype(jnp.bfloat16)

    t_tile = _T_TILE
    num_tiles = T // t_tile
    b_block = B // 2 if (B % 32 == 0) else B
    num_b = B // b_block

    out, h_n, c_n = pl.pallas_call(
        _lstm_tile_kernel,
        grid=(num_b, num_tiles),
        in_specs=[
            pl.BlockSpec((b_block, t_tile), lambda b, t: (b, t)),
            pl.BlockSpec((_KA, 4 * H), lambda b, t: (0, 0)),
        ],
        out_specs=(
            pl.BlockSpec(memory_space=pl.ANY),
            pl.BlockSpec((b_block, H), lambda b, t: (b, 0)),
            pl.BlockSpec((b_block, H), lambda b, t: (b, 0)),
        ),
        out_shape=(
            jax.ShapeDtypeStruct((B, T, H), jnp.float32),
            jax.ShapeDtypeStruct((B, H), jnp.float32),
            jax.ShapeDtypeStruct((B, H), jnp.float32),
        ),
        scratch_shapes=[
            pltpu.VMEM((b_block, _U * H), jnp.float32),
            pltpu.VMEM((2, b_block, _KA), jnp.bfloat16),
            pltpu.VMEM((2, b_block, _U, H), jnp.float32),
            pltpu.SemaphoreType.DMA((2,)),
        ],
        compiler_params=pltpu.CompilerParams(
            dimension_semantics=("parallel", "arbitrary")),
    )(x.astype(jnp.float32), waug)

    return out, (h_n[None, ...], c_n[None, ...])


# speedup vs baseline: 1.2074x; 1.2074x over previous
import jax
import jax.numpy as jnp
from jax.experimental import pallas as pl
from jax.experimental.pallas import tpu as pltpu

_T_TILE = 128
_U = 8
_NS = 4
_KA = 256


def _lstm_tile_kernel(x_ref, waug_ref, out_ref, hn_ref, cn_ref, scr_ref):
    Bb, H = hn_ref.shape
    H3 = 3 * H
    sr = Bb // _NS
    mrb_per_stream = sr // 4
    tid = pl.program_id(1)

    @pl.when(tid == 0)
    def _init():
        hn_ref[...] = jnp.zeros_like(hn_ref)
        cn_ref[...] = jnp.zeros_like(cn_ref)

    waug = waug_ref[...]
    xb = x_ref[...].astype(jnp.bfloat16)
    ones_col = jnp.ones((sr, 1), jnp.bfloat16)
    zpad = jnp.zeros((sr, _KA - H - 2), jnp.bfloat16)

    pltpu.matmul_push_rhs(waug[:, 0:256], staging_register=0, mxu_index=0)
    pltpu.matmul_push_rhs(waug[:, 256:512], staging_register=0, mxu_index=1)

    @pl.when(tid == 0)
    def _clear_mrb():
        for k in range(_NS):
            for m in range(2):
                pltpu.matmul_pop(acc_addr=k * mrb_per_stream,
                                 shape=(sr, 256), dtype=jnp.float32,
                                 mxu_index=m)

    def cell(x_col, h_bf, c, k, first):
        aug = jnp.concatenate([h_bf, x_col, ones_col, zpad], axis=1)
        addr = k * mrb_per_stream
        lsr = 0 if first else None
        pltpu.matmul_acc_lhs(acc_addr=addr, lhs=aug, mxu_index=0,
                             load_staged_rhs=lsr)
        pltpu.matmul_acc_lhs(acc_addr=addr, lhs=aug, mxu_index=1,
                             load_staged_rhs=lsr)
        g0 = pltpu.matmul_pop(acc_addr=addr, shape=(sr, 256),
                              dtype=jnp.float32, mxu_index=0)
        g1 = pltpu.matmul_pop(acc_addr=addr, shape=(sr, 256),
                              dtype=jnp.float32, mxu_index=1)
        gates = jnp.concatenate([g0, g1], axis=1)
        sig = jnp.tanh(gates[:, :H3]) * 0.5 + 0.5
        g_gate = jnp.tanh(gates[:, H3:])
        c = sig[:, H:2 * H] * c + sig[:, :H] * g_gate
        h32 = sig[:, 2 * H:H3] * jnp.tanh(c)
        return h32, h32.astype(jnp.bfloat16), c

    hs = [hn_ref[k * sr:(k + 1) * sr, :].astype(jnp.bfloat16)
          for k in range(_NS)]
    cs = [cn_ref[k * sr:(k + 1) * sr, :] for k in range(_NS)]
    hs32 = [None] * _NS

    for ci in range(_T_TILE // _U):
        for j in range(_U):
            t = ci * _U + j
            for k in range(_NS):
                hs32[k], hs[k], cs[k] = cell(
                    xb[k * sr:(k + 1) * sr, t:t + 1], hs[k], cs[k], k,
                    first=(t == 0 and k == 0))
                scr_ref[k * sr:(k + 1) * sr, j * H:(j + 1) * H] = hs32[k]
        rg = min(32, Bb)
        for r in range(0, Bb, rg):
            out_ref[r:r + rg, ci * _U:(ci + 1) * _U, :] = (
                scr_ref[r:r + rg, :].reshape(rg, _U, H))

    for k in range(_NS):
        hn_ref[k * sr:(k + 1) * sr, :] = hs32[k]
        cn_ref[k * sr:(k + 1) * sr, :] = cs[k]


def kernel(x, w_ih, w_hh, b_ih, b_hh):
    B, T = x.shape
    H = w_hh.shape[1]

    def perm_gates(a, axis):
        i, f, g, o = jnp.split(a.astype(jnp.float32), 4, axis=axis)
        return jnp.concatenate([i, f, o, g], axis=axis)

    whh_t = perm_gates(jnp.transpose(w_hh), axis=1)
    wih_row = perm_gates(w_ih.reshape(1, 4 * H), axis=1)
    bias = perm_gates((b_ih + b_hh).reshape(1, 4 * H), axis=1)
    waug = jnp.concatenate(
        [whh_t, wih_row, bias,
         jnp.zeros((_KA - H - 2, 4 * H), jnp.float32)], axis=0)
    col_scale = jnp.concatenate([jnp.full((1, 3 * H), 0.5, jnp.float32),
                                 jnp.ones((1, H), jnp.float32)], axis=1)
    waug = (waug * col_scale).astype(jnp.bfloat16)

    t_tile = _T_TILE
    num_tiles = T // t_tile
    b_block = B // 2 if (B % 32 == 0) else B
    num_b = B // b_block

    out, h_n, c_n = pl.pallas_call(
        _lstm_tile_kernel,
        grid=(num_b, num_tiles),
        in_specs=[
            pl.BlockSpec((b_block, t_tile), lambda b, t: (b, t)),
            pl.BlockSpec((_KA, 4 * H), lambda b, t: (0, 0)),
        ],
        out_specs=(
            pl.BlockSpec((b_block, t_tile, H), lambda b, t: (b, t, 0)),
            pl.BlockSpec((b_block, H), lambda b, t: (b, 0)),
            pl.BlockSpec((b_block, H), lambda b, t: (b, 0)),
        ),
        out_shape=(
            jax.ShapeDtypeStruct((B, T, H), jnp.float32),
            jax.ShapeDtypeStruct((B, H), jnp.float32),
            jax.ShapeDtypeStruct((B, H), jnp.float32),
        ),
        scratch_shapes=[pltpu.VMEM((b_block, _U * H), jnp.float32)],
        compiler_params=pltpu.CompilerParams(
            dimension_semantics=("parallel", "arbitrary")),
    )(x.astype(jnp.float32), waug)

    return out, (h_n[None, ...], c_n[None, ...])


# speedup vs baseline: 1.2361x; 1.0238x over previous
import jax
import jax.numpy as jnp
from jax.experimental import pallas as pl
from jax.experimental.pallas import tpu as pltpu

_T_TILE = 128
_U = 8
_NS = 4


def _lstm_tile_kernel(x_ref, waug_ref, out_ref, hn_ref, cn_ref, scr_ref):
    Bb, H = hn_ref.shape
    H3 = 3 * H
    sr = Bb // _NS
    tid = pl.program_id(1)

    @pl.when(tid == 0)
    def _init():
        hn_ref[...] = jnp.zeros_like(hn_ref)
        cn_ref[...] = jnp.zeros_like(cn_ref)

    waug = waug_ref[...]
    xb = x_ref[...].astype(jnp.bfloat16)
    ones_col = jnp.ones((sr, 1), jnp.bfloat16)

    def cell(x_col, h_bf, c):
        aug = jnp.concatenate([h_bf, x_col, ones_col], axis=1)
        gates = jnp.dot(aug, waug, preferred_element_type=jnp.float32)
        sig = jnp.tanh(gates[:, :H3]) * 0.5 + 0.5
        g_gate = jnp.tanh(gates[:, H3:])
        c = sig[:, H:2 * H] * c + sig[:, :H] * g_gate
        h32 = sig[:, 2 * H:H3] * jnp.tanh(c)
        return h32, h32.astype(jnp.bfloat16), c

    hs = [hn_ref[k * sr:(k + 1) * sr, :].astype(jnp.bfloat16)
          for k in range(_NS)]
    cs = [cn_ref[k * sr:(k + 1) * sr, :] for k in range(_NS)]
    hs32 = [None] * _NS

    for ci in range(_T_TILE // _U):
        for j in range(_U):
            t = ci * _U + j
            for k in range(_NS):
                hs32[k], hs[k], cs[k] = cell(
                    xb[k * sr:(k + 1) * sr, t:t + 1], hs[k], cs[k])
                scr_ref[k * sr:(k + 1) * sr, j * H:(j + 1) * H] = hs32[k]
        rg = min(32, Bb)
        for r in range(0, Bb, rg):
            out_ref[r:r + rg, ci * _U:(ci + 1) * _U, :] = (
                scr_ref[r:r + rg, :].reshape(rg, _U, H))

    for k in range(_NS):
        hn_ref[k * sr:(k + 1) * sr, :] = hs32[k]
        cn_ref[k * sr:(k + 1) * sr, :] = cs[k]


def kernel(x, w_ih, w_hh, b_ih, b_hh):
    B, T = x.shape
    H = w_hh.shape[1]

    def perm_gates(a, axis):
        i, f, g, o = jnp.split(a.astype(jnp.float32), 4, axis=axis)
        return jnp.concatenate([i, f, o, g], axis=axis)

    whh_t = perm_gates(jnp.transpose(w_hh), axis=1)
    wih_row = perm_gates(w_ih.reshape(1, 4 * H), axis=1)
    bias = perm_gates((b_ih + b_hh).reshape(1, 4 * H), axis=1)
    waug = jnp.concatenate([whh_t, wih_row, bias], axis=0)
    col_scale = jnp.concatenate([jnp.full((1, 3 * H), 0.5, jnp.float32),
                                 jnp.ones((1, H), jnp.float32)], axis=1)
    waug = (waug * col_scale).astype(jnp.bfloat16)

    t_tile = _T_TILE
    num_tiles = T // t_tile
    b_block = B // 2 if (B % 32 == 0) else B
    num_b = B // b_block

    out, h_n, c_n = pl.pallas_call(
        _lstm_tile_kernel,
        grid=(num_b, num_tiles),
        in_specs=[
            pl.BlockSpec((b_block, t_tile), lambda b, t: (b, t)),
            pl.BlockSpec((H + 2, 4 * H), lambda b, t: (0, 0)),
        ],
        out_specs=(
            pl.BlockSpec((b_block, t_tile, H), lambda b, t: (b, t, 0)),
            pl.BlockSpec((b_block, H), lambda b, t: (b, 0)),
            pl.BlockSpec((b_block, H), lambda b, t: (b, 0)),
        ),
        out_shape=(
            jax.ShapeDtypeStruct((B, T, H), jnp.float32),
            jax.ShapeDtypeStruct((B, H), jnp.float32),
            jax.ShapeDtypeStruct((B, H), jnp.float32),
        ),
        scratch_shapes=[pltpu.VMEM((b_block, _U * H), jnp.float32)],
        compiler_params=pltpu.CompilerParams(
            dimension_semantics=("parallel", "arbitrary")),
    )(x.astype(jnp.float32), waug)

    return out, (h_n[None, ...], c_n[None, ...])
